# Initial kernel scaffold; baseline (speedup 1.0000x reference)
import jax
import jax.numpy as jnp
from jax.experimental import pallas as pl


def kernel(X, emb, W1, b1, W2, b2, Wout, bout):
    raise NotImplementedError("write your pallas kernel here")


# speedup vs baseline: 3.1486x; 3.1486x over previous
import functools

import jax
import jax.numpy as jnp
from jax.experimental import pallas as pl
from jax.experimental.pallas import tpu as pltpu

_FT = 1024
_VT = 3200
_R = 8


def _gather_body(tok_ref, emb_ref, out_ref):
    del tok_ref
    out_ref[...] = emb_ref[...]


def _mlp_body(h0_ref, w1_ref, b1_ref, w2_ref, b2_ref, out_ref, h_s, acc, *,
              n_blocks, n_ff):
    i = pl.program_id(0)
    j = pl.program_id(1)

    @pl.when(jnp.logical_and(i == 0, j == 0))
    def _():
        h_s[...] = h0_ref[...]

    @pl.when(j == 0)
    def _():
        acc[...] = h_s[...]

    h = h_s[...]
    mid = jnp.maximum(
        jnp.dot(h, w1_ref[0], preferred_element_type=jnp.float32,
                precision=jax.lax.Precision.HIGHEST) + b1_ref[0], 0.0)
    acc[...] += jnp.dot(mid, w2_ref[0], preferred_element_type=jnp.float32,
                        precision=jax.lax.Precision.HIGHEST)

    @pl.when(j == n_ff - 1)
    def _():
        h_s[...] = acc[...] + b2_ref[0]

    @pl.when(jnp.logical_and(i == n_blocks - 1, j == n_ff - 1))
    def _():
        out_ref[...] = acc[...] + b2_ref[0]


def _proj_body(h_ref, wout_ref, bout_ref, out_ref):
    res = jnp.dot(h_ref[...], wout_ref[...], preferred_element_type=jnp.float32,
                  precision=jax.lax.Precision.HIGHEST) + bout_ref[...]
    out_ref[...] = res[0:4]


def kernel(X, emb, W1, b1, W2, b2, Wout, bout):
    batch, _ = X.shape
    vocab, d_model = emb.shape
    n_blocks, _, d_ff = W1.shape
    n_ff = d_ff // _FT
    n_vt = vocab // _VT
    sub = d_model // 128

    tokens = jnp.pad(X[:, -1], (0, _R - batch))

    emb3 = emb.reshape(vocab, sub, 128)
    h0 = pl.pallas_call(
        _gather_body,
        grid_spec=pltpu.PrefetchScalarGridSpec(
            num_scalar_prefetch=1,
            grid=(_R,),
            in_specs=[pl.BlockSpec((1, sub, 128), lambda t, tok: (tok[t], 0, 0))],
            out_specs=pl.BlockSpec((1, sub, 128), lambda t, tok: (t, 0, 0)),
        ),
        out_shape=jax.ShapeDtypeStruct((_R, sub, 128), jnp.float32),
    )(tokens, emb3).reshape(_R, d_model)

    h_final = pl.pallas_call(
        functools.partial(_mlp_body, n_blocks=n_blocks, n_ff=n_ff),
        grid=(n_blocks, n_ff),
        in_specs=[
            pl.BlockSpec((_R, d_model), lambda i, j: (0, 0)),
            pl.BlockSpec((1, d_model, _FT), lambda i, j: (i, 0, j)),
            pl.BlockSpec((1, 1, _FT), lambda i, j: (i, 0, j)),
            pl.BlockSpec((1, _FT, d_model), lambda i, j: (i, j, 0)),
            pl.BlockSpec((1, 1, d_model), lambda i, j: (i, 0, 0)),
        ],
        out_specs=pl.BlockSpec((_R, d_model), lambda i, j: (0, 0)),
        out_shape=jax.ShapeDtypeStruct((_R, d_model), jnp.float32),
        scratch_shapes=[
            pltpu.VMEM((_R, d_model), jnp.float32),
            pltpu.VMEM((_R, d_model), jnp.float32),
        ],
        compiler_params=pltpu.CompilerParams(
            dimension_semantics=("arbitrary", "arbitrary")),
    )(h0, W1, b1.reshape(n_blocks, 1, d_ff), W2, b2.reshape(n_blocks, 1, d_model))

    logits = pl.pallas_call(
        _proj_body,
        grid=(n_vt,),
        in_specs=[
            pl.BlockSpec((_R, d_model), lambda v: (0, 0)),
            pl.BlockSpec((d_model, _VT), lambda v: (0, v)),
            pl.BlockSpec((1, _VT), lambda v: (0, v)),
        ],
        out_specs=pl.BlockSpec((batch, _VT), lambda v: (0, v)),
        out_shape=jax.ShapeDtypeStruct((batch, vocab), jnp.float32),
        compiler_params=pltpu.CompilerParams(
            dimension_semantics=("arbitrary",)),
    )(h_final, Wout, bout.reshape(1, vocab))

    return logits


# speedup vs baseline: 5.7017x; 1.8109x over previous
import functools

import jax
import jax.numpy as jnp
from jax.experimental import pallas as pl
from jax.experimental.pallas import tpu as pltpu

_FT = 1024
_VT = 3200
_R = 8


def _mlp_body(tok_ref, emb_ref, w1_ref, b1_ref, w2_ref, b2_ref, out_ref,
              h_s, acc, gsem, *, n_blocks, n_ff):
    i = pl.program_id(0)
    j = pl.program_id(1)

    @pl.when(jnp.logical_and(i == 0, j == 0))
    def _():
        for r in range(_R):
            pltpu.make_async_copy(
                emb_ref.at[pl.ds(tok_ref[r], 1), :],
                h_s.at[pl.ds(r, 1), :],
                gsem.at[r],
            ).start()
        for r in range(_R):
            pltpu.make_async_copy(
                emb_ref.at[pl.ds(tok_ref[r], 1), :],
                h_s.at[pl.ds(r, 1), :],
                gsem.at[r],
            ).wait()

    @pl.when(j == 0)
    def _():
        acc[...] = h_s[...]

    h = h_s[...]
    mid = jnp.maximum(
        jnp.dot(h, w1_ref[0], preferred_element_type=jnp.float32,
                precision=jax.lax.Precision.HIGHEST) + b1_ref[0], 0.0)
    acc[...] += jnp.dot(mid, w2_ref[0], preferred_element_type=jnp.float32,
                        precision=jax.lax.Precision.HIGHEST)

    @pl.when(j == n_ff - 1)
    def _():
        h_s[...] = acc[...] + b2_ref[0]

    @pl.when(jnp.logical_and(i == n_blocks - 1, j == n_ff - 1))
    def _():
        out_ref[...] = acc[...] + b2_ref[0]


def _proj_body(h_ref, wout_ref, bout_ref, out_ref):
    res = jnp.dot(h_ref[...], wout_ref[...], preferred_element_type=jnp.float32,
                  precision=jax.lax.Precision.HIGHEST) + bout_ref[...]
    out_ref[...] = res[0:4]


def kernel(X, emb, W1, b1, W2, b2, Wout, bout):
    batch, _ = X.shape
    vocab, d_model = emb.shape
    n_blocks, _, d_ff = W1.shape
    n_ff = d_ff // _FT
    n_vt = vocab // _VT

    tokens = jnp.pad(X[:, -1], (0, _R - batch))

    h_final = pl.pallas_call(
        functools.partial(_mlp_body, n_blocks=n_blocks, n_ff=n_ff),
        grid_spec=pltpu.PrefetchScalarGridSpec(
            num_scalar_prefetch=1,
            grid=(n_blocks, n_ff),
            in_specs=[
                pl.BlockSpec(memory_space=pl.ANY),
                pl.BlockSpec((1, d_model, _FT), lambda i, j, tok: (i, 0, j)),
                pl.BlockSpec((1, 1, _FT), lambda i, j, tok: (i, 0, j)),
                pl.BlockSpec((1, _FT, d_model), lambda i, j, tok: (i, j, 0)),
                pl.BlockSpec((1, 1, d_model), lambda i, j, tok: (i, 0, 0)),
            ],
            out_specs=pl.BlockSpec((_R, d_model), lambda i, j, tok: (0, 0)),
            scratch_shapes=[
                pltpu.VMEM((_R, d_model), jnp.float32),
                pltpu.VMEM((_R, d_model), jnp.float32),
                pltpu.SemaphoreType.DMA((_R,)),
            ],
        ),
        out_shape=jax.ShapeDtypeStruct((_R, d_model), jnp.float32),
        compiler_params=pltpu.CompilerParams(
            dimension_semantics=("arbitrary", "arbitrary")),
    )(tokens, emb, W1, b1.reshape(n_blocks, 1, d_ff), W2,
      b2.reshape(n_blocks, 1, d_model))

    logits = pl.pallas_call(
        _proj_body,
        grid=(n_vt,),
        in_specs=[
            pl.BlockSpec((_R, d_model), lambda v: (0, 0)),
            pl.BlockSpec((d_model, _VT), lambda v: (0, v)),
            pl.BlockSpec((1, _VT), lambda v: (0, v)),
        ],
        out_specs=pl.BlockSpec((batch, _VT), lambda v: (0, v)),
        out_shape=jax.ShapeDtypeStruct((batch, vocab), jnp.float32),
        compiler_params=pltpu.CompilerParams(
            dimension_semantics=("arbitrary",)),
    )(h_final, Wout, bout.reshape(1, vocab))

    return logits


# speedup vs baseline: 8.4651x; 1.4846x over previous
import functools

import jax
import jax.numpy as jnp
from jax.experimental import pallas as pl
from jax.experimental.pallas import tpu as pltpu

_FT = 1024
_VT = 3200
_R = 8


def _mlp_body(tok_ref, emb_ref, w1_ref, b1_ref, w2_ref, b2_ref, out_ref,
              h_s, acc, gsem, *, n_blocks, n_ff):
    i = pl.program_id(0)
    j = pl.program_id(1)

    @pl.when(jnp.logical_and(i == 0, j == 0))
    def _():
        for r in range(_R):
            pltpu.make_async_copy(
                emb_ref.at[pl.ds(tok_ref[r], 1), :],
                h_s.at[pl.ds(r, 1), :],
                gsem.at[r],
            ).start()
        for r in range(_R):
            pltpu.make_async_copy(
                emb_ref.at[pl.ds(tok_ref[r], 1), :],
                h_s.at[pl.ds(r, 1), :],
                gsem.at[r],
            ).wait()

    @pl.when(j == 0)
    def _():
        acc[...] = h_s[...]

    h = h_s[...]
    mid = jnp.maximum(
        jnp.dot(h, w1_ref[0], preferred_element_type=jnp.float32) + b1_ref[0],
        0.0)
    acc[...] += jnp.dot(mid, w2_ref[0], preferred_element_type=jnp.float32)

    @pl.when(j == n_ff - 1)
    def _():
        h_s[...] = acc[...] + b2_ref[0]

    @pl.when(jnp.logical_and(i == n_blocks - 1, j == n_ff - 1))
    def _():
        out_ref[...] = acc[...] + b2_ref[0]


def _proj_body(h_ref, wout_ref, bout_ref, out_ref):
    res = jnp.dot(h_ref[...], wout_ref[...],
                  preferred_element_type=jnp.float32) + bout_ref[...]
    out_ref[...] = res[0:4]


def kernel(X, emb, W1, b1, W2, b2, Wout, bout):
    batch, _ = X.shape
    vocab, d_model = emb.shape
    n_blocks, _, d_ff = W1.shape
    n_ff = d_ff // _FT
    n_vt = vocab // _VT

    tokens = jnp.pad(X[:, -1], (0, _R - batch))

    h_final = pl.pallas_call(
        functools.partial(_mlp_body, n_blocks=n_blocks, n_ff=n_ff),
        grid_spec=pltpu.PrefetchScalarGridSpec(
            num_scalar_prefetch=1,
            grid=(n_blocks, n_ff),
            in_specs=[
                pl.BlockSpec(memory_space=pl.ANY),
                pl.BlockSpec((1, d_model, _FT), lambda i, j, tok: (i, 0, j)),
                pl.BlockSpec((1, 1, _FT), lambda i, j, tok: (i, 0, j)),
                pl.BlockSpec((1, _FT, d_model), lambda i, j, tok: (i, j, 0)),
                pl.BlockSpec((1, 1, d_model), lambda i, j, tok: (i, 0, 0)),
            ],
            out_specs=pl.BlockSpec((_R, d_model), lambda i, j, tok: (0, 0)),
            scratch_shapes=[
                pltpu.VMEM((_R, d_model), jnp.float32),
                pltpu.VMEM((_R, d_model), jnp.float32),
                pltpu.SemaphoreType.DMA((_R,)),
            ],
        ),
        out_shape=jax.ShapeDtypeStruct((_R, d_model), jnp.float32),
        compiler_params=pltpu.CompilerParams(
            dimension_semantics=("arbitrary", "arbitrary")),
    )(tokens, emb, W1, b1.reshape(n_blocks, 1, d_ff), W2,
      b2.reshape(n_blocks, 1, d_model))

    logits = pl.pallas_call(
        _proj_body,
        grid=(n_vt,),
        in_specs=[
            pl.BlockSpec((_R, d_model), lambda v: (0, 0)),
            pl.BlockSpec((d_model, _VT), lambda v: (0, v)),
            pl.BlockSpec((1, _VT), lambda v: (0, v)),
        ],
        out_specs=pl.BlockSpec((batch, _VT), lambda v: (0, v)),
        out_shape=jax.ShapeDtypeStruct((batch, vocab), jnp.float32),
        compiler_params=pltpu.CompilerParams(
            dimension_semantics=("arbitrary",)),
    )(h_final, Wout, bout.reshape(1, vocab))

    return logits


# speedup vs baseline: 8.6275x; 1.0192x over previous
import functools

import jax
import jax.numpy as jnp
from jax.experimental import pallas as pl
from jax.experimental.pallas import tpu as pltpu

_FT = 1024
_VT = 3200
_R = 8


def _body(tok_ref, emb_ref, w1_ref, b1_ref, w2_ref, b2_ref, wout_ref, bout_ref,
          out_ref, h_s, acc, gsem, *, n_blocks, n_ff, n_vt, batch):
    t = pl.program_id(0)
    mlp_steps = n_blocks * n_ff
    j = t % n_ff

    @pl.when(t == 0)
    def _():
        for r in range(_R):
            pltpu.make_async_copy(
                emb_ref.at[pl.ds(tok_ref[r], 1), :],
                h_s.at[pl.ds(r, 1), :],
                gsem.at[r],
            ).start()
        for r in range(_R):
            pltpu.make_async_copy(
                emb_ref.at[pl.ds(tok_ref[r], 1), :],
                h_s.at[pl.ds(r, 1), :],
                gsem.at[r],
            ).wait()

    @pl.when(t < mlp_steps)
    def _():
        @pl.when(j == 0)
        def _():
            acc[...] = h_s[...]

        h = h_s[...]
        mid = jnp.maximum(
            jnp.dot(h, w1_ref[0], preferred_element_type=jnp.float32)
            + b1_ref[0], 0.0)
        acc[...] += jnp.dot(mid, w2_ref[0], preferred_element_type=jnp.float32)

        @pl.when(j == n_ff - 1)
        def _():
            h_s[...] = acc[...] + b2_ref[0]

    @pl.when(t >= mlp_steps)
    def _():
        res = jnp.dot(h_s[...], wout_ref[...],
                      preferred_element_type=jnp.float32) + bout_ref[...]
        out_ref[...] = res[0:batch]


def kernel(X, emb, W1, b1, W2, b2, Wout, bout):
    batch, _ = X.shape
    vocab, d_model = emb.shape
    n_blocks, _, d_ff = W1.shape
    n_ff = d_ff // _FT
    n_vt = vocab // _VT
    mlp_steps = n_blocks * n_ff

    tokens = jnp.pad(X[:, -1], (0, _R - batch))

    def w_i(t, tok):
        return jnp.minimum(t, mlp_steps - 1) // n_ff

    def w_j(t, tok):
        return jnp.minimum(t, mlp_steps - 1) % n_ff

    def v_i(t, tok):
        return jnp.maximum(t - mlp_steps, 0)

    logits = pl.pallas_call(
        functools.partial(_body, n_blocks=n_blocks, n_ff=n_ff, n_vt=n_vt,
                          batch=batch),
        grid_spec=pltpu.PrefetchScalarGridSpec(
            num_scalar_prefetch=1,
            grid=(mlp_steps + n_vt,),
            in_specs=[
                pl.BlockSpec(memory_space=pl.ANY),
                pl.BlockSpec((1, d_model, _FT), lambda t, tok: (w_i(t, tok), 0, w_j(t, tok))),
                pl.BlockSpec((1, 1, _FT), lambda t, tok: (w_i(t, tok), 0, w_j(t, tok))),
                pl.BlockSpec((1, _FT, d_model), lambda t, tok: (w_i(t, tok), w_j(t, tok), 0)),
                pl.BlockSpec((1, 1, d_model), lambda t, tok: (w_i(t, tok), 0, 0)),
                pl.BlockSpec((d_model, _VT), lambda t, tok: (0, v_i(t, tok))),
                pl.BlockSpec((1, _VT), lambda t, tok: (0, v_i(t, tok))),
            ],
            out_specs=pl.BlockSpec((batch, _VT), lambda t, tok: (0, v_i(t, tok))),
            scratch_shapes=[
                pltpu.VMEM((_R, d_model), jnp.float32),
                pltpu.VMEM((_R, d_model), jnp.float32),
                pltpu.SemaphoreType.DMA((_R,)),
            ],
        ),
        out_shape=jax.ShapeDtypeStruct((batch, vocab), jnp.float32),
        compiler_params=pltpu.CompilerParams(
            dimension_semantics=("arbitrary",),
            vmem_limit_bytes=120 * 1024 * 1024),
    )(tokens, emb, W1, b1.reshape(n_blocks, 1, d_ff), W2,
      b2.reshape(n_blocks, 1, d_model), Wout, bout.reshape(1, vocab))

    return logits
